# Initial kernel scaffold; baseline (speedup 1.0000x reference)
import jax
import jax.numpy as jnp
from jax.experimental import pallas as pl


def kernel(z, w_in, b_in, codebook, w_out, b_out):
    raise NotImplementedError("write your pallas kernel here")


# speedup vs baseline: 2.1961x; 2.1961x over previous
import jax
import jax.numpy as jnp
from jax.experimental import pallas as pl
from jax.experimental.pallas import tpu as pltpu

_BETA = 0.25
_INT_MAX = 2**31 - 1


def _stats_kernel(z_ref, w_in_ref, b_in_ref, cb_ref,
                  idx_ref, sumzq_ref, sumsq_ref,
                  bestv_s, besti_s, sumsq_s):
    i = pl.program_id(0)
    nt = pl.num_programs(0)
    bn = z_ref.shape[0]
    pages = cb_ref.shape[0]

    x = jax.lax.dot_general(z_ref[...], w_in_ref[...],
                            (((1,), (1,)), ((), ())),
                            preferred_element_type=jnp.float32)
    x = x + b_in_ref[...]
    zq = x / jnp.sqrt(jnp.sum(x * x, axis=1, keepdims=True))

    cb = cb_ref[...]
    cn = cb / jnp.sqrt(jnp.sum(cb * cb, axis=1, keepdims=True))

    s2 = jnp.sum(zq * zq, axis=1, keepdims=True)
    c2 = jnp.sum(cn * cn, axis=1, keepdims=True)
    dots = jax.lax.dot_general(zq, cn, (((1,), (1,)), ((), ())),
                               preferred_element_type=jnp.float32)
    d = s2 - 2.0 * dots + c2.reshape(1, pages)

    m = jnp.min(d)
    flat_ids = (jax.lax.broadcasted_iota(jnp.int32, d.shape, 0) * pages
                + jax.lax.broadcasted_iota(jnp.int32, d.shape, 1))
    fid = (jnp.min(jnp.where(d == m, flat_ids, jnp.int32(_INT_MAX)))
           + i * (bn * pages))

    szq = jnp.sum(zq, axis=0, keepdims=True)
    ssq = jnp.sum(s2)

    @pl.when(i == 0)
    def _():
        bestv_s[0, 0] = m
        besti_s[0, 0] = fid
        sumzq_ref[...] = szq
        sumsq_s[0, 0] = ssq

    @pl.when(i > 0)
    def _():
        better = m < bestv_s[0, 0]
        bestv_s[0, 0] = jnp.where(better, m, bestv_s[0, 0])
        besti_s[0, 0] = jnp.where(better, fid, besti_s[0, 0])
        sumzq_ref[...] = sumzq_ref[...] + szq
        sumsq_s[0, 0] = sumsq_s[0, 0] + ssq

    @pl.when(i == nt - 1)
    def _():
        idx_ref[0, 0] = besti_s[0, 0]
        sumsq_ref[0, 0] = sumsq_s[0, 0]


def _emit_kernel(idx_ref, sumsq_ref, cb_ref, w_out_ref, b_out_ref, sumzq_ref,
                 out_ref, loss_ref):
    pages = cb_ref.shape[0]
    n_total = out_ref.shape[0] * pl.num_programs(0)
    codes = cb_ref.shape[1]

    ic = jnp.clip(idx_ref[0, 0], 0, pages - 1)
    row = cb_ref[pl.ds(ic, 1), :]
    code = row / jnp.sqrt(jnp.sum(row * row))
    rowvec = jax.lax.dot_general(code, w_out_ref[...],
                                 (((1,), (1,)), ((), ())),
                                 preferred_element_type=jnp.float32)
    rowvec = rowvec + b_out_ref[...]
    out_ref[...] = jnp.broadcast_to(rowvec, out_ref.shape)

    @pl.when(pl.program_id(0) == 0)
    def _():
        c2 = jnp.sum(code * code)
        cross = jnp.sum(sumzq_ref[...] * code)
        mse = (sumsq_ref[0, 0] - 2.0 * cross + n_total * c2) / (n_total * codes)
        loss_ref[0, 0] = (1.0 + _BETA) * mse


def kernel(z, w_in, b_in, codebook, w_out, b_out):
    n, features = z.shape
    codes = w_in.shape[0]
    pages = codebook.shape[0]
    bn = 1024
    nt = n // bn

    b_in2 = b_in.reshape(1, codes)
    b_out2 = b_out.reshape(1, features)

    idx, sumzq, sumsq = pl.pallas_call(
        _stats_kernel,
        grid=(nt,),
        in_specs=[
            pl.BlockSpec((bn, features), lambda i: (i, 0)),
            pl.BlockSpec((codes, features), lambda i: (0, 0)),
            pl.BlockSpec((1, codes), lambda i: (0, 0)),
            pl.BlockSpec((pages, codes), lambda i: (0, 0)),
        ],
        out_specs=[
            pl.BlockSpec(memory_space=pltpu.SMEM),
            pl.BlockSpec((1, codes), lambda i: (0, 0)),
            pl.BlockSpec(memory_space=pltpu.SMEM),
        ],
        out_shape=[
            jax.ShapeDtypeStruct((1, 1), jnp.int32),
            jax.ShapeDtypeStruct((1, codes), jnp.float32),
            jax.ShapeDtypeStruct((1, 1), jnp.float32),
        ],
        scratch_shapes=[
            pltpu.SMEM((1, 1), jnp.float32),
            pltpu.SMEM((1, 1), jnp.int32),
            pltpu.SMEM((1, 1), jnp.float32),
        ],
    )(z, w_in, b_in2, codebook)

    out, loss = pl.pallas_call(
        _emit_kernel,
        grid=(nt,),
        in_specs=[
            pl.BlockSpec(memory_space=pltpu.SMEM),
            pl.BlockSpec(memory_space=pltpu.SMEM),
            pl.BlockSpec((pages, codes), lambda i: (0, 0)),
            pl.BlockSpec((features, codes), lambda i: (0, 0)),
            pl.BlockSpec((1, features), lambda i: (0, 0)),
            pl.BlockSpec((1, codes), lambda i: (0, 0)),
        ],
        out_specs=[
            pl.BlockSpec((bn, features), lambda i: (i, 0)),
            pl.BlockSpec(memory_space=pltpu.SMEM),
        ],
        out_shape=[
            jax.ShapeDtypeStruct((n, features), jnp.float32),
            jax.ShapeDtypeStruct((1, 1), jnp.float32),
        ],
    )(idx, sumsq, codebook, w_out, b_out2, sumzq)

    return (out, loss[0, 0], idx[0, 0])


# speedup vs baseline: 2.7371x; 1.2463x over previous
import jax
import jax.numpy as jnp
from jax.experimental import pallas as pl
from jax.experimental.pallas import tpu as pltpu

_BETA = 0.25
_INT_MAX = 2**31 - 1


def _stats_kernel(z_ref, w_in_ref, b_in_ref, cb_ref,
                  row_ref, sumzq_ref, sumsq_ref,
                  bestv_s, bestr_s, sumsq_s):
    i = pl.program_id(0)
    nt = pl.num_programs(0)
    bn = z_ref.shape[0]
    pages = cb_ref.shape[0]

    x = jax.lax.dot_general(z_ref[...], w_in_ref[...],
                            (((1,), (1,)), ((), ())),
                            preferred_element_type=jnp.float32)
    x = x + b_in_ref[...]
    zq = x / jnp.sqrt(jnp.sum(x * x, axis=1, keepdims=True))

    cb = cb_ref[...]
    cn = cb / jnp.sqrt(jnp.sum(cb * cb, axis=1, keepdims=True))

    s2 = jnp.sum(zq * zq, axis=1, keepdims=True)
    c2 = jnp.sum(cn * cn, axis=1, keepdims=True)
    dots = jax.lax.dot_general(zq, cn, (((1,), (1,)), ((), ())),
                               preferred_element_type=jnp.float32)
    d = s2 - 2.0 * dots + c2.reshape(1, pages)

    cm = d[:, 0:128]
    for c0 in range(128, pages, 128):
        cm = jnp.minimum(cm, d[:, c0:c0 + 128])
    rowmin = jnp.min(cm, axis=1, keepdims=True)

    m = jnp.min(rowmin)
    rows = jax.lax.broadcasted_iota(jnp.int32, rowmin.shape, 0)
    rloc = (jnp.min(jnp.where(rowmin == m, rows, jnp.int32(_INT_MAX)))
            + i * bn)

    szq = jnp.sum(zq, axis=0, keepdims=True)
    ssq = jnp.sum(s2)

    @pl.when(i == 0)
    def _():
        bestv_s[0, 0] = m
        bestr_s[0, 0] = rloc
        sumzq_ref[...] = szq
        sumsq_s[0, 0] = ssq

    @pl.when(i > 0)
    def _():
        better = m < bestv_s[0, 0]
        bestv_s[0, 0] = jnp.where(better, m, bestv_s[0, 0])
        bestr_s[0, 0] = jnp.where(better, rloc, bestr_s[0, 0])
        sumzq_ref[...] = sumzq_ref[...] + szq
        sumsq_s[0, 0] = sumsq_s[0, 0] + ssq

    @pl.when(i == nt - 1)
    def _():
        row_ref[0, 0] = bestr_s[0, 0]
        sumsq_ref[0, 0] = sumsq_s[0, 0]


def _emit_kernel(row_s, z_ref, w_in_ref, b_in_ref, cb_ref, w_out_ref,
                 b_out_ref, sumzq_ref, sumsq_ref,
                 out_ref, loss_ref, idx_ref,
                 rowvec_s):
    pages = cb_ref.shape[0]
    codes = cb_ref.shape[1]
    n_total = out_ref.shape[0] * pl.num_programs(0)

    @pl.when(pl.program_id(0) == 0)
    def _():
        r = row_s[0]
        xr = jax.lax.dot_general(z_ref[pl.ds(r % 8, 1), :], w_in_ref[...],
                                 (((1,), (1,)), ((), ())),
                                 preferred_element_type=jnp.float32)
        xr = xr + b_in_ref[...]
        zq = xr / jnp.sqrt(jnp.sum(xr * xr))
        cb = cb_ref[...]
        cn = cb / jnp.sqrt(jnp.sum(cb * cb, axis=1, keepdims=True))
        s2 = jnp.sum(zq * zq)
        c2 = jnp.sum(cn * cn, axis=1, keepdims=True)
        dots = jax.lax.dot_general(zq, cn, (((1,), (1,)), ((), ())),
                                   preferred_element_type=jnp.float32)
        drow = s2 - 2.0 * dots + c2.reshape(1, pages)
        dm = jnp.min(drow)
        cols = jax.lax.broadcasted_iota(jnp.int32, drow.shape, 1)
        col = jnp.min(jnp.where(drow == dm, cols, jnp.int32(_INT_MAX)))
        idx = r * pages + col
        idx_ref[0, 0] = idx

        ic = jnp.clip(idx, 0, pages - 1)
        crow = cb_ref[pl.ds(ic, 1), :]
        code = crow / jnp.sqrt(jnp.sum(crow * crow))
        rowvec = jax.lax.dot_general(code, w_out_ref[...],
                                     (((1,), (1,)), ((), ())),
                                     preferred_element_type=jnp.float32)
        rowvec_s[...] = rowvec + b_out_ref[...]

        cc2 = jnp.sum(code * code)
        cross = jnp.sum(sumzq_ref[...] * code)
        mse = (sumsq_ref[0, 0] - 2.0 * cross + n_total * cc2) / (n_total * codes)
        loss_ref[0, 0] = (1.0 + _BETA) * mse

    out_ref[...] = jnp.broadcast_to(rowvec_s[...], out_ref.shape)


def kernel(z, w_in, b_in, codebook, w_out, b_out):
    n, features = z.shape
    codes = w_in.shape[0]
    pages = codebook.shape[0]
    bn = 1024
    nt = n // bn

    b_in2 = b_in.reshape(1, codes)
    b_out2 = b_out.reshape(1, features)

    row, sumzq, sumsq = pl.pallas_call(
        _stats_kernel,
        grid=(nt,),
        in_specs=[
            pl.BlockSpec((bn, features), lambda i: (i, 0)),
            pl.BlockSpec((codes, features), lambda i: (0, 0)),
            pl.BlockSpec((1, codes), lambda i: (0, 0)),
            pl.BlockSpec((pages, codes), lambda i: (0, 0)),
        ],
        out_specs=[
            pl.BlockSpec(memory_space=pltpu.SMEM),
            pl.BlockSpec((1, codes), lambda i: (0, 0)),
            pl.BlockSpec(memory_space=pltpu.SMEM),
        ],
        out_shape=[
            jax.ShapeDtypeStruct((1, 1), jnp.int32),
            jax.ShapeDtypeStruct((1, codes), jnp.float32),
            jax.ShapeDtypeStruct((1, 1), jnp.float32),
        ],
        scratch_shapes=[
            pltpu.SMEM((1, 1), jnp.float32),
            pltpu.SMEM((1, 1), jnp.int32),
            pltpu.SMEM((1, 1), jnp.float32),
        ],
    )(z, w_in, b_in2, codebook)

    grid_spec = pltpu.PrefetchScalarGridSpec(
        num_scalar_prefetch=1,
        grid=(nt,),
        in_specs=[
            pl.BlockSpec((8, features), lambda i, row_s: (row_s[0] // 8, 0)),
            pl.BlockSpec((codes, features), lambda i, row_s: (0, 0)),
            pl.BlockSpec((1, codes), lambda i, row_s: (0, 0)),
            pl.BlockSpec((pages, codes), lambda i, row_s: (0, 0)),
            pl.BlockSpec((features, codes), lambda i, row_s: (0, 0)),
            pl.BlockSpec((1, features), lambda i, row_s: (0, 0)),
            pl.BlockSpec((1, codes), lambda i, row_s: (0, 0)),
            pl.BlockSpec(memory_space=pltpu.SMEM),
        ],
        out_specs=[
            pl.BlockSpec((bn, features), lambda i, row_s: (i, 0)),
            pl.BlockSpec(memory_space=pltpu.SMEM),
            pl.BlockSpec(memory_space=pltpu.SMEM),
        ],
        scratch_shapes=[
            pltpu.VMEM((1, features), jnp.float32),
        ],
    )
    out, loss, idx = pl.pallas_call(
        _emit_kernel,
        grid_spec=grid_spec,
        out_shape=[
            jax.ShapeDtypeStruct((n, features), jnp.float32),
            jax.ShapeDtypeStruct((1, 1), jnp.float32),
            jax.ShapeDtypeStruct((1, 1), jnp.int32),
        ],
    )(row.reshape(1), z, w_in, b_in2, codebook, w_out, b_out2, sumzq, sumsq)

    return (out, loss[0, 0], idx[0, 0])


# speedup vs baseline: 2.7674x; 1.0111x over previous
import jax
import jax.numpy as jnp
from jax.experimental import pallas as pl
from jax.experimental.pallas import tpu as pltpu

_BETA = 0.25
_INT_MAX = 2**31 - 1


def _stats_kernel(z_ref, w_in_ref, b_in_ref, cb_ref,
                  row_ref, sumzq_ref, sumsq_ref,
                  bestv_s, bestr_s, sumsq_s):
    i = pl.program_id(0)
    nt = pl.num_programs(0)
    bn = z_ref.shape[0]
    pages = cb_ref.shape[0]

    x = jax.lax.dot_general(z_ref[...], w_in_ref[...],
                            (((1,), (1,)), ((), ())),
                            preferred_element_type=jnp.float32)
    x = x + b_in_ref[...]
    zq = x * jax.lax.rsqrt(jnp.sum(x * x, axis=1, keepdims=True))

    cb = cb_ref[...]
    cn = cb * jax.lax.rsqrt(jnp.sum(cb * cb, axis=1, keepdims=True))

    s2 = jnp.sum(zq * zq, axis=1, keepdims=True)
    c2 = jnp.sum(cn * cn, axis=1, keepdims=True)
    dots = jax.lax.dot_general(zq, cn, (((1,), (1,)), ((), ())),
                               preferred_element_type=jnp.float32)
    d = s2 - 2.0 * dots + c2.reshape(1, pages)

    cm = d[:, 0:128]
    for c0 in range(128, pages, 128):
        cm = jnp.minimum(cm, d[:, c0:c0 + 128])
    rowmin = jnp.min(cm, axis=1, keepdims=True)

    m = jnp.min(rowmin)
    rows = jax.lax.broadcasted_iota(jnp.int32, rowmin.shape, 0)
    rloc = (jnp.min(jnp.where(rowmin == m, rows, jnp.int32(_INT_MAX)))
            + i * bn)

    szq = jnp.sum(zq, axis=0, keepdims=True)
    ssq = jnp.sum(s2)

    @pl.when(i == 0)
    def _():
        bestv_s[0, 0] = m
        bestr_s[0, 0] = rloc
        sumzq_ref[...] = szq
        sumsq_s[0, 0] = ssq

    @pl.when(i > 0)
    def _():
        better = m < bestv_s[0, 0]
        bestv_s[0, 0] = jnp.where(better, m, bestv_s[0, 0])
        bestr_s[0, 0] = jnp.where(better, rloc, bestr_s[0, 0])
        sumzq_ref[...] = sumzq_ref[...] + szq
        sumsq_s[0, 0] = sumsq_s[0, 0] + ssq

    @pl.when(i == nt - 1)
    def _():
        row_ref[0, 0] = bestr_s[0, 0]
        sumsq_ref[0, 0] = sumsq_s[0, 0]


def _emit_kernel(row_s, z_ref, w_in_ref, b_in_ref, cb_ref, w_out_ref,
                 b_out_ref, sumzq_ref, sumsq_ref,
                 out_ref, loss_ref, idx_ref,
                 rowvec_s):
    pages = cb_ref.shape[0]
    codes = cb_ref.shape[1]
    n_total = out_ref.shape[0] * pl.num_programs(0)

    @pl.when(pl.program_id(0) == 0)
    def _():
        r = row_s[0]
        xr = jax.lax.dot_general(z_ref[pl.ds(r % 8, 1), :], w_in_ref[...],
                                 (((1,), (1,)), ((), ())),
                                 preferred_element_type=jnp.float32)
        xr = xr + b_in_ref[...]
        zq = xr / jnp.sqrt(jnp.sum(xr * xr))
        cb = cb_ref[...]
        cn = cb / jnp.sqrt(jnp.sum(cb * cb, axis=1, keepdims=True))
        s2 = jnp.sum(zq * zq)
        c2 = jnp.sum(cn * cn, axis=1, keepdims=True)
        dots = jax.lax.dot_general(zq, cn, (((1,), (1,)), ((), ())),
                                   preferred_element_type=jnp.float32)
        drow = s2 - 2.0 * dots + c2.reshape(1, pages)
        dm = jnp.min(drow)
        cols = jax.lax.broadcasted_iota(jnp.int32, drow.shape, 1)
        col = jnp.min(jnp.where(drow == dm, cols, jnp.int32(_INT_MAX)))
        idx = r * pages + col
        idx_ref[0, 0] = idx

        ic = jnp.clip(idx, 0, pages - 1)
        crow = cb_ref[pl.ds(ic, 1), :]
        code = crow / jnp.sqrt(jnp.sum(crow * crow))
        rowvec = jax.lax.dot_general(code, w_out_ref[...],
                                     (((1,), (1,)), ((), ())),
                                     preferred_element_type=jnp.float32)
        rowvec_s[...] = rowvec + b_out_ref[...]

        cc2 = jnp.sum(code * code)
        cross = jnp.sum(sumzq_ref[...] * code)
        mse = (sumsq_ref[0, 0] - 2.0 * cross + n_total * cc2) / (n_total * codes)
        loss_ref[0, 0] = (1.0 + _BETA) * mse

    out_ref[...] = jnp.broadcast_to(rowvec_s[...], out_ref.shape)


def kernel(z, w_in, b_in, codebook, w_out, b_out):
    n, features = z.shape
    codes = w_in.shape[0]
    pages = codebook.shape[0]
    bn = 1024
    nt = n // bn

    b_in2 = b_in.reshape(1, codes)
    b_out2 = b_out.reshape(1, features)

    row, sumzq, sumsq = pl.pallas_call(
        _stats_kernel,
        grid=(nt,),
        in_specs=[
            pl.BlockSpec((bn, features), lambda i: (i, 0)),
            pl.BlockSpec((codes, features), lambda i: (0, 0)),
            pl.BlockSpec((1, codes), lambda i: (0, 0)),
            pl.BlockSpec((pages, codes), lambda i: (0, 0)),
        ],
        out_specs=[
            pl.BlockSpec(memory_space=pltpu.SMEM),
            pl.BlockSpec((1, codes), lambda i: (0, 0)),
            pl.BlockSpec(memory_space=pltpu.SMEM),
        ],
        out_shape=[
            jax.ShapeDtypeStruct((1, 1), jnp.int32),
            jax.ShapeDtypeStruct((1, codes), jnp.float32),
            jax.ShapeDtypeStruct((1, 1), jnp.float32),
        ],
        scratch_shapes=[
            pltpu.SMEM((1, 1), jnp.float32),
            pltpu.SMEM((1, 1), jnp.int32),
            pltpu.SMEM((1, 1), jnp.float32),
        ],
    )(z, w_in, b_in2, codebook)

    grid_spec = pltpu.PrefetchScalarGridSpec(
        num_scalar_prefetch=1,
        grid=(nt,),
        in_specs=[
            pl.BlockSpec((8, features), lambda i, row_s: (row_s[0] // 8, 0)),
            pl.BlockSpec((codes, features), lambda i, row_s: (0, 0)),
            pl.BlockSpec((1, codes), lambda i, row_s: (0, 0)),
            pl.BlockSpec((pages, codes), lambda i, row_s: (0, 0)),
            pl.BlockSpec((features, codes), lambda i, row_s: (0, 0)),
            pl.BlockSpec((1, features), lambda i, row_s: (0, 0)),
            pl.BlockSpec((1, codes), lambda i, row_s: (0, 0)),
            pl.BlockSpec(memory_space=pltpu.SMEM),
        ],
        out_specs=[
            pl.BlockSpec((bn, features), lambda i, row_s: (i, 0)),
            pl.BlockSpec(memory_space=pltpu.SMEM),
            pl.BlockSpec(memory_space=pltpu.SMEM),
        ],
        scratch_shapes=[
            pltpu.VMEM((1, features), jnp.float32),
        ],
    )
    out, loss, idx = pl.pallas_call(
        _emit_kernel,
        grid_spec=grid_spec,
        out_shape=[
            jax.ShapeDtypeStruct((n, features), jnp.float32),
            jax.ShapeDtypeStruct((1, 1), jnp.float32),
            jax.ShapeDtypeStruct((1, 1), jnp.int32),
        ],
    )(row.reshape(1), z, w_in, b_in2, codebook, w_out, b_out2, sumzq, sumsq)

    return (out, loss[0, 0], idx[0, 0])


# speedup vs baseline: 2.7934x; 1.0094x over previous
import jax
import jax.numpy as jnp
from jax.experimental import pallas as pl
from jax.experimental.pallas import tpu as pltpu

_BETA = 0.25
_INT_MAX = 2**31 - 1


def _stats_kernel(z_ref, w_in_ref, b_in_ref, cb_ref,
                  row_ref, sumzq_ref, sumsq_ref,
                  bestv_s, bestr_s, sumsq_s):
    i = pl.program_id(0)
    nt = pl.num_programs(0)
    bn = z_ref.shape[0]
    pages = cb_ref.shape[0]

    x = jax.lax.dot_general(z_ref[...], w_in_ref[...],
                            (((1,), (1,)), ((), ())),
                            preferred_element_type=jnp.float32)
    x = x + b_in_ref[...]
    zq = x * jax.lax.rsqrt(jnp.sum(x * x, axis=1, keepdims=True))

    cb = cb_ref[...]
    cn = cb * jax.lax.rsqrt(jnp.sum(cb * cb, axis=1, keepdims=True))

    s2 = jnp.sum(zq * zq, axis=1, keepdims=True)
    c2 = jnp.sum(cn * cn, axis=1, keepdims=True)
    dots = jax.lax.dot_general(zq, cn, (((1,), (1,)), ((), ())),
                               preferred_element_type=jnp.float32)
    d = s2 - 2.0 * dots + c2.reshape(1, pages)

    cm = d[:, 0:128]
    for c0 in range(128, pages, 128):
        cm = jnp.minimum(cm, d[:, c0:c0 + 128])
    rowmin = jnp.min(cm, axis=1, keepdims=True)

    m = jnp.min(rowmin)
    rows = jax.lax.broadcasted_iota(jnp.int32, rowmin.shape, 0)
    rloc = (jnp.min(jnp.where(rowmin == m, rows, jnp.int32(_INT_MAX)))
            + i * bn)

    szq = jnp.sum(zq, axis=0, keepdims=True)
    ssq = jnp.sum(s2)

    @pl.when(i == 0)
    def _():
        bestv_s[0, 0] = m
        bestr_s[0, 0] = rloc
        sumzq_ref[...] = szq
        sumsq_s[0, 0] = ssq

    @pl.when(i > 0)
    def _():
        better = m < bestv_s[0, 0]
        bestv_s[0, 0] = jnp.where(better, m, bestv_s[0, 0])
        bestr_s[0, 0] = jnp.where(better, rloc, bestr_s[0, 0])
        sumzq_ref[...] = sumzq_ref[...] + szq
        sumsq_s[0, 0] = sumsq_s[0, 0] + ssq

    @pl.when(i == nt - 1)
    def _():
        row_ref[0, 0] = bestr_s[0, 0]
        sumsq_ref[0, 0] = sumsq_s[0, 0]


def _emit_kernel(row_s, z_ref, w_in_ref, b_in_ref, cb_ref, w_out_ref,
                 b_out_ref, sumzq_ref, sumsq_ref,
                 out_ref, loss_ref, idx_ref,
                 rowvec_s):
    pages = cb_ref.shape[0]
    codes = cb_ref.shape[1]
    n_total = out_ref.shape[0] * pl.num_programs(0)

    @pl.when(pl.program_id(0) == 0)
    def _():
        r = row_s[0]
        xr = jax.lax.dot_general(z_ref[pl.ds(r % 8, 1), :], w_in_ref[...],
                                 (((1,), (1,)), ((), ())),
                                 preferred_element_type=jnp.float32)
        xr = xr + b_in_ref[...]
        zq = xr / jnp.sqrt(jnp.sum(xr * xr))
        cb = cb_ref[...]
        cn = cb / jnp.sqrt(jnp.sum(cb * cb, axis=1, keepdims=True))
        s2 = jnp.sum(zq * zq)
        c2 = jnp.sum(cn * cn, axis=1, keepdims=True)
        dots = jax.lax.dot_general(zq, cn, (((1,), (1,)), ((), ())),
                                   preferred_element_type=jnp.float32)
        drow = s2 - 2.0 * dots + c2.reshape(1, pages)
        dm = jnp.min(drow)
        cols = jax.lax.broadcasted_iota(jnp.int32, drow.shape, 1)
        col = jnp.min(jnp.where(drow == dm, cols, jnp.int32(_INT_MAX)))
        idx = r * pages + col
        idx_ref[0, 0] = idx

        ic = jnp.clip(idx, 0, pages - 1)
        crow = cb_ref[pl.ds(ic, 1), :]
        code = crow / jnp.sqrt(jnp.sum(crow * crow))
        rowvec = jax.lax.dot_general(code, w_out_ref[...],
                                     (((1,), (1,)), ((), ())),
                                     preferred_element_type=jnp.float32)
        rowvec_s[...] = rowvec + b_out_ref[...]

        cc2 = jnp.sum(code * code)
        cross = jnp.sum(sumzq_ref[...] * code)
        mse = (sumsq_ref[0, 0] - 2.0 * cross + n_total * cc2) / (n_total * codes)
        loss_ref[0, 0] = (1.0 + _BETA) * mse

    out_ref[...] = jnp.broadcast_to(rowvec_s[...], out_ref.shape)


def kernel(z, w_in, b_in, codebook, w_out, b_out):
    n, features = z.shape
    codes = w_in.shape[0]
    pages = codebook.shape[0]
    bn = 2304
    nt = n // bn

    b_in2 = b_in.reshape(1, codes)
    b_out2 = b_out.reshape(1, features)

    row, sumzq, sumsq = pl.pallas_call(
        _stats_kernel,
        grid=(nt,),
        in_specs=[
            pl.BlockSpec((bn, features), lambda i: (i, 0)),
            pl.BlockSpec((codes, features), lambda i: (0, 0)),
            pl.BlockSpec((1, codes), lambda i: (0, 0)),
            pl.BlockSpec((pages, codes), lambda i: (0, 0)),
        ],
        out_specs=[
            pl.BlockSpec(memory_space=pltpu.SMEM),
            pl.BlockSpec((1, codes), lambda i: (0, 0)),
            pl.BlockSpec(memory_space=pltpu.SMEM),
        ],
        out_shape=[
            jax.ShapeDtypeStruct((1, 1), jnp.int32),
            jax.ShapeDtypeStruct((1, codes), jnp.float32),
            jax.ShapeDtypeStruct((1, 1), jnp.float32),
        ],
        scratch_shapes=[
            pltpu.SMEM((1, 1), jnp.float32),
            pltpu.SMEM((1, 1), jnp.int32),
            pltpu.SMEM((1, 1), jnp.float32),
        ],
    )(z, w_in, b_in2, codebook)

    grid_spec = pltpu.PrefetchScalarGridSpec(
        num_scalar_prefetch=1,
        grid=(nt,),
        in_specs=[
            pl.BlockSpec((8, features), lambda i, row_s: (row_s[0] // 8, 0)),
            pl.BlockSpec((codes, features), lambda i, row_s: (0, 0)),
            pl.BlockSpec((1, codes), lambda i, row_s: (0, 0)),
            pl.BlockSpec((pages, codes), lambda i, row_s: (0, 0)),
            pl.BlockSpec((features, codes), lambda i, row_s: (0, 0)),
            pl.BlockSpec((1, features), lambda i, row_s: (0, 0)),
            pl.BlockSpec((1, codes), lambda i, row_s: (0, 0)),
            pl.BlockSpec(memory_space=pltpu.SMEM),
        ],
        out_specs=[
            pl.BlockSpec((bn, features), lambda i, row_s: (i, 0)),
            pl.BlockSpec(memory_space=pltpu.SMEM),
            pl.BlockSpec(memory_space=pltpu.SMEM),
        ],
        scratch_shapes=[
            pltpu.VMEM((1, features), jnp.float32),
        ],
    )
    out, loss, idx = pl.pallas_call(
        _emit_kernel,
        grid_spec=grid_spec,
        out_shape=[
            jax.ShapeDtypeStruct((n, features), jnp.float32),
            jax.ShapeDtypeStruct((1, 1), jnp.float32),
            jax.ShapeDtypeStruct((1, 1), jnp.int32),
        ],
    )(row.reshape(1), z, w_in, b_in2, codebook, w_out, b_out2, sumzq, sumsq)

    return (out, loss[0, 0], idx[0, 0])


# speedup vs baseline: 3.1217x; 1.1175x over previous
import jax
import jax.numpy as jnp
from jax.experimental import pallas as pl
from jax.experimental.pallas import tpu as pltpu

_BETA = 0.25
_INT_MAX = 2**31 - 1


def _vq_kernel(z_ref, w_in_ref, b_in_ref, cb_ref, w_out_ref, b_out_ref,
               out_ref, loss_ref, idx_ref,
               zq_s, rowvec_s, sumzq_s, bestv_s, bestr_s, sumsq_s):
    i = pl.program_id(0)
    nt = pl.num_programs(0) // 2
    bn = z_ref.shape[0]
    pages = cb_ref.shape[0]
    codes = cb_ref.shape[1]
    n_total = bn * nt

    cb = cb_ref[...]
    cn = cb * jax.lax.rsqrt(jnp.sum(cb * cb, axis=1, keepdims=True))
    c2 = jnp.sum(cn * cn, axis=1, keepdims=True)

    @pl.when(i < nt)
    def _scan():
        x = jax.lax.dot_general(z_ref[...], w_in_ref[...],
                                (((1,), (1,)), ((), ())),
                                preferred_element_type=jnp.float32)
        x = x + b_in_ref[...]
        zq = x * jax.lax.rsqrt(jnp.sum(x * x, axis=1, keepdims=True))
        zq_s[pl.ds(i * bn, bn), :] = zq

        s2 = jnp.sum(zq * zq, axis=1, keepdims=True)
        zq_aug = jnp.concatenate([zq, s2, jnp.ones_like(s2)], axis=1)
        cn_aug = jnp.concatenate([-2.0 * cn, jnp.ones_like(c2), c2], axis=1)
        d = jax.lax.dot_general(zq_aug, cn_aug, (((1,), (1,)), ((), ())),
                                preferred_element_type=jnp.float32)

        cm = d[:, 0:128]
        for c0 in range(128, pages, 128):
            cm = jnp.minimum(cm, d[:, c0:c0 + 128])
        rowmin = jnp.min(cm, axis=1, keepdims=True)

        m = jnp.min(rowmin)
        rows = jax.lax.broadcasted_iota(jnp.int32, rowmin.shape, 0)
        rloc = (jnp.min(jnp.where(rowmin == m, rows, jnp.int32(_INT_MAX)))
                + i * bn)

        szq = jnp.sum(zq, axis=0, keepdims=True)
        ssq = jnp.sum(s2)

        @pl.when(i == 0)
        def _():
            bestv_s[0, 0] = m
            bestr_s[0, 0] = rloc
            sumzq_s[...] = szq
            sumsq_s[0, 0] = ssq

        @pl.when(i > 0)
        def _():
            better = m < bestv_s[0, 0]
            bestv_s[0, 0] = jnp.where(better, m, bestv_s[0, 0])
            bestr_s[0, 0] = jnp.where(better, rloc, bestr_s[0, 0])
            sumzq_s[...] = sumzq_s[...] + szq
            sumsq_s[0, 0] = sumsq_s[0, 0] + ssq

    @pl.when(i == nt)
    def _finalize():
        r = bestr_s[0, 0]
        zq_row = zq_s[pl.ds(r, 1), :]
        s2r = jnp.sum(zq_row * zq_row)
        dots = jax.lax.dot_general(zq_row, cn, (((1,), (1,)), ((), ())),
                                   preferred_element_type=jnp.float32)
        drow = s2r - 2.0 * dots + c2.reshape(1, pages)
        dm = jnp.min(drow)
        cols = jax.lax.broadcasted_iota(jnp.int32, drow.shape, 1)
        col = jnp.min(jnp.where(drow == dm, cols, jnp.int32(_INT_MAX)))
        idx = r * pages + col
        idx_ref[0, 0] = idx

        ic = jnp.clip(idx, 0, pages - 1)
        crow = cb_ref[pl.ds(ic, 1), :]
        code = crow / jnp.sqrt(jnp.sum(crow * crow))
        rowvec = jax.lax.dot_general(code, w_out_ref[...],
                                     (((1,), (1,)), ((), ())),
                                     preferred_element_type=jnp.float32)
        rowvec_s[...] = rowvec + b_out_ref[...]

        cc2 = jnp.sum(code * code)
        cross = jnp.sum(sumzq_s[...] * code)
        mse = (sumsq_s[0, 0] - 2.0 * cross + n_total * cc2) / (n_total * codes)
        loss_ref[0, 0] = (1.0 + _BETA) * mse

    @pl.when(i >= nt)
    def _emit():
        out_ref[...] = jnp.broadcast_to(rowvec_s[...], out_ref.shape)


def kernel(z, w_in, b_in, codebook, w_out, b_out):
    n, features = z.shape
    codes = w_in.shape[0]
    pages = codebook.shape[0]
    bn = 2304
    nt = n // bn

    b_in2 = b_in.reshape(1, codes)
    b_out2 = b_out.reshape(1, features)

    out, loss, idx = pl.pallas_call(
        _vq_kernel,
        grid=(2 * nt,),
        in_specs=[
            pl.BlockSpec((bn, features), lambda i: (jnp.minimum(i, nt - 1), 0)),
            pl.BlockSpec((codes, features), lambda i: (0, 0)),
            pl.BlockSpec((1, codes), lambda i: (0, 0)),
            pl.BlockSpec((pages, codes), lambda i: (0, 0)),
            pl.BlockSpec((features, codes), lambda i: (0, 0)),
            pl.BlockSpec((1, features), lambda i: (0, 0)),
        ],
        out_specs=[
            pl.BlockSpec((bn, features),
                         lambda i: (jnp.maximum(i - nt, 0), 0)),
            pl.BlockSpec(memory_space=pltpu.SMEM),
            pl.BlockSpec(memory_space=pltpu.SMEM),
        ],
        out_shape=[
            jax.ShapeDtypeStruct((n, features), jnp.float32),
            jax.ShapeDtypeStruct((1, 1), jnp.float32),
            jax.ShapeDtypeStruct((1, 1), jnp.int32),
        ],
        scratch_shapes=[
            pltpu.VMEM((n, codes), jnp.float32),
            pltpu.VMEM((1, features), jnp.float32),
            pltpu.VMEM((1, codes), jnp.float32),
            pltpu.SMEM((1, 1), jnp.float32),
            pltpu.SMEM((1, 1), jnp.int32),
            pltpu.SMEM((1, 1), jnp.float32),
        ],
    )(z, w_in, b_in2, codebook, w_out, b_out2)

    return (out, loss[0, 0], idx[0, 0])


# speedup vs baseline: 3.2750x; 1.0491x over previous
import jax
import jax.numpy as jnp
from jax.experimental import pallas as pl
from jax.experimental.pallas import tpu as pltpu

_BETA = 0.25
_INT_MAX = 2**31 - 1


def _vq_kernel(z_ref, w_in_ref, b_in_ref, cb_ref, w_out_ref, b_out_ref,
               out_ref, loss_ref, idx_ref,
               zq_s, rowvec_s, sumzq_s, bestv_s, bestr_s, sumsq_s):
    i = pl.program_id(0)
    nt = pl.num_programs(0) // 2
    bn = z_ref.shape[0]
    pages = cb_ref.shape[0]
    codes = cb_ref.shape[1]
    n_total = bn * nt

    @pl.when(i < nt)
    def _scan():
        cb = cb_ref[...]
        cn = cb * jax.lax.rsqrt(jnp.sum(cb * cb, axis=1, keepdims=True))
        x = jax.lax.dot_general(z_ref[...], w_in_ref[...],
                                (((1,), (1,)), ((), ())),
                                preferred_element_type=jnp.float32)
        x = x + b_in_ref[...]
        zq = x * jax.lax.rsqrt(jnp.sum(x * x, axis=1, keepdims=True))
        zq_s[pl.ds(i * bn, bn), :] = zq

        s2 = jnp.sum(zq * zq, axis=1, keepdims=True)

        dots = jax.lax.dot_general(zq, cn, (((1,), (1,)), ((), ())),
                                   preferred_element_type=jnp.float32)

        cm = dots[:, 0:128]
        for c0 in range(128, pages, 128):
            cm = jnp.maximum(cm, dots[:, c0:c0 + 128])
        rowmax = jnp.max(cm, axis=1, keepdims=True)

        m = jnp.max(rowmax)
        rows = jax.lax.broadcasted_iota(jnp.int32, rowmax.shape, 0)
        rloc = (jnp.min(jnp.where(rowmax == m, rows, jnp.int32(_INT_MAX)))
                + i * bn)

        szq = jnp.sum(zq, axis=0, keepdims=True)
        ssq = jnp.sum(s2)

        @pl.when(i == 0)
        def _():
            bestv_s[0, 0] = m
            bestr_s[0, 0] = rloc
            sumzq_s[...] = szq
            sumsq_s[0, 0] = ssq

        @pl.when(i > 0)
        def _():
            better = m > bestv_s[0, 0]
            bestv_s[0, 0] = jnp.where(better, m, bestv_s[0, 0])
            bestr_s[0, 0] = jnp.where(better, rloc, bestr_s[0, 0])
            sumzq_s[...] = sumzq_s[...] + szq
            sumsq_s[0, 0] = sumsq_s[0, 0] + ssq

    @pl.when(i == nt)
    def _finalize():
        cb = cb_ref[...]
        cn = cb * jax.lax.rsqrt(jnp.sum(cb * cb, axis=1, keepdims=True))
        c2 = jnp.sum(cn * cn, axis=1, keepdims=True)
        r = bestr_s[0, 0]
        zq_row = zq_s[pl.ds(r, 1), :]
        s2r = jnp.sum(zq_row * zq_row)
        dots = jax.lax.dot_general(zq_row, cn, (((1,), (1,)), ((), ())),
                                   preferred_element_type=jnp.float32)
        drow = s2r - 2.0 * dots + c2.reshape(1, pages)
        dm = jnp.min(drow)
        cols = jax.lax.broadcasted_iota(jnp.int32, drow.shape, 1)
        col = jnp.min(jnp.where(drow == dm, cols, jnp.int32(_INT_MAX)))
        idx = r * pages + col
        idx_ref[0, 0] = idx

        ic = jnp.clip(idx, 0, pages - 1)
        crow = cb_ref[pl.ds(ic, 1), :]
        code = crow / jnp.sqrt(jnp.sum(crow * crow))
        rowvec = jax.lax.dot_general(code, w_out_ref[...],
                                     (((1,), (1,)), ((), ())),
                                     preferred_element_type=jnp.float32)
        rowvec_s[...] = rowvec + b_out_ref[...]

        cc2 = jnp.sum(code * code)
        cross = jnp.sum(sumzq_s[...] * code)
        mse = (sumsq_s[0, 0] - 2.0 * cross + n_total * cc2) / (n_total * codes)
        loss_ref[0, 0] = (1.0 + _BETA) * mse

    @pl.when(i >= nt)
    def _emit():
        out_ref[...] = jnp.broadcast_to(rowvec_s[...], out_ref.shape)


def kernel(z, w_in, b_in, codebook, w_out, b_out):
    n, features = z.shape
    codes = w_in.shape[0]
    pages = codebook.shape[0]
    bn = 2304
    nt = n // bn

    b_in2 = b_in.reshape(1, codes)
    b_out2 = b_out.reshape(1, features)

    out, loss, idx = pl.pallas_call(
        _vq_kernel,
        grid=(2 * nt,),
        in_specs=[
            pl.BlockSpec((bn, features), lambda i: (jnp.minimum(i, nt - 1), 0)),
            pl.BlockSpec((codes, features), lambda i: (0, 0)),
            pl.BlockSpec((1, codes), lambda i: (0, 0)),
            pl.BlockSpec((pages, codes), lambda i: (0, 0)),
            pl.BlockSpec((features, codes), lambda i: (0, 0)),
            pl.BlockSpec((1, features), lambda i: (0, 0)),
        ],
        out_specs=[
            pl.BlockSpec((bn, features),
                         lambda i: (jnp.maximum(i - nt, 0), 0)),
            pl.BlockSpec(memory_space=pltpu.SMEM),
            pl.BlockSpec(memory_space=pltpu.SMEM),
        ],
        out_shape=[
            jax.ShapeDtypeStruct((n, features), jnp.float32),
            jax.ShapeDtypeStruct((1, 1), jnp.float32),
            jax.ShapeDtypeStruct((1, 1), jnp.int32),
        ],
        scratch_shapes=[
            pltpu.VMEM((n, codes), jnp.float32),
            pltpu.VMEM((1, features), jnp.float32),
            pltpu.VMEM((1, codes), jnp.float32),
            pltpu.SMEM((1, 1), jnp.float32),
            pltpu.SMEM((1, 1), jnp.int32),
            pltpu.SMEM((1, 1), jnp.float32),
        ],
    )(z, w_in, b_in2, codebook, w_out, b_out2)

    return (out, loss[0, 0], idx[0, 0])
